# baseline (speedup 1.0000x reference)
import functools

import jax
import jax.numpy as jnp
from jax import lax
from jax.experimental import pallas as pl
from jax.experimental.pallas import tpu as pltpu
from jax.experimental.pallas import tpu_sc as plsc

N = 10000
E = 320000
D = 128

NC = 2
NS = 16
NW = NC * NS
CHUNK = 50
NSLOT = 4
NCROWS = E // CHUNK
GPW = NCROWS // NW
BLK = 8
NBLOCKS = GPW // BLK

RPT = 624
TAIL = N - NS * RPT
ZROWS = 48

ROW_BLOCK = 1000


def _sc_neigh_sum(features, ei):
    mesh = plsc.VectorSubcoreMesh(
        core_axis_name="c", subcore_axis_name="s",
        num_cores=NC, num_subcores=NS)

    @functools.partial(
        pl.kernel,
        out_type=jax.ShapeDtypeStruct((NC * N, D), jnp.float32),
        mesh=mesh,
        scratch_types=[
            pltpu.VMEM((2, BLK, CHUNK), jnp.int32),
            pltpu.VMEM((2, BLK, CHUNK), jnp.int32),
            pltpu.VMEM((NSLOT, CHUNK, D), jnp.float32),
            pltpu.VMEM_SHARED((N, D), jnp.float32),
            pltpu.SemaphoreType.DMA,
            pltpu.SemaphoreType.DMA,
        ],
    )
    def k(feat_hbm, ei_hbm, out_hbm,
          sbuf, dbuf, rows_v, accum, gsem, ssem):
        cid = lax.axis_index("c")
        sid = lax.axis_index("s")
        wid = sid * NC + cid

        def zrow(i, carry):
            for c in range(D // 16):
                rows_v[0, i, pl.ds(c * 16, 16)] = jnp.zeros((16,),
                                                            jnp.float32)
            return carry

        lax.fori_loop(0, ZROWS, zrow, 0)
        r0 = pl.multiple_of(sid * RPT, 8)
        zsrc = rows_v.at[0, pl.ds(0, ZROWS)]
        for q in range(RPT // ZROWS):
            pltpu.async_copy(zsrc, accum.at[pl.ds(r0 + q * ZROWS, ZROWS)],
                             ssem)

        @pl.when(sid == 0)
        def _():
            pltpu.async_copy(rows_v.at[0, pl.ds(0, TAIL)],
                             accum.at[pl.ds(NS * RPT, TAIL)], ssem)

        for q in range(RPT // ZROWS):
            pltpu.make_async_copy(zsrc,
                                  accum.at[pl.ds(r0 + q * ZROWS, ZROWS)],
                                  ssem).wait()

        @pl.when(sid == 0)
        def _():
            pltpu.make_async_copy(rows_v.at[0, pl.ds(0, TAIL)],
                                  accum.at[pl.ds(NS * RPT, TAIL)],
                                  ssem).wait()

        plsc.subcore_barrier()

        base = wid * GPW

        def fire_gather(s, bsel, q):
            pltpu.async_copy(feat_hbm.at[sbuf.at[bsel, q]],
                             rows_v.at[s], gsem)

        def drain_gather(s):
            pltpu.make_async_copy(feat_hbm.at[sbuf.at[0, 0]],
                                  rows_v.at[s], gsem).wait()

        def fire_scatter(s, bsel, q):
            pltpu.async_copy(rows_v.at[s], accum.at[dbuf.at[bsel, q]],
                             ssem, add=True)

        def drain_scatter(s):
            pltpu.make_async_copy(rows_v.at[s], accum.at[dbuf.at[0, 0]],
                                  ssem).wait()

        pltpu.sync_copy(ei_hbm.at[0, pl.ds(pl.multiple_of(base, 8), BLK)],
                        sbuf.at[0])
        pltpu.sync_copy(ei_hbm.at[1, pl.ds(pl.multiple_of(base, 8), BLK)],
                        dbuf.at[0])
        for q in range(NSLOT):
            fire_gather(q, 0, q)

        def body(h, carry):
            nb = h + 1

            @pl.when(nb < NBLOCKS)
            def _():
                bsel_n = lax.rem(nb, 2)
                roff = pl.multiple_of(base + nb * BLK, 8)
                pltpu.sync_copy(ei_hbm.at[0, pl.ds(roff, BLK)],
                                sbuf.at[bsel_n])
                pltpu.sync_copy(ei_hbm.at[1, pl.ds(roff, BLK)],
                                dbuf.at[bsel_n])

            bcur = lax.rem(h, 2)
            bnext = lax.rem(h + 1, 2)
            for qq in range(BLK):
                s = qq % NSLOT
                drain_gather(s)
                fire_scatter(s, bcur, qq)
                drain_scatter(s)
                nxt_q = (qq + NSLOT) % BLK
                nxt_b = bcur if qq < BLK - NSLOT else bnext

                @pl.when(BLK * h + qq + NSLOT < GPW)
                def _(s=s, nxt_b=nxt_b, nxt_q=nxt_q):
                    fire_gather(s, nxt_b, nxt_q)

            return carry

        lax.fori_loop(0, NBLOCKS, body, 0)
        plsc.subcore_barrier()

        w0 = pl.multiple_of(cid * N + sid * RPT, 8)
        pltpu.sync_copy(accum.at[pl.ds(r0, RPT)], out_hbm.at[pl.ds(w0, RPT)])

        @pl.when(sid == 0)
        def _():
            pltpu.sync_copy(accum.at[pl.ds(NS * RPT, TAIL)],
                            out_hbm.at[pl.ds(cid * N + NS * RPT, TAIL)])

    return k(features, ei)


def _tc_finish(parts, features, W1, W2):
    def body(p_ref, x_ref, w1_ref, w2_ref, o_ref):
        ns = p_ref[0] + p_ref[1]
        x = x_ref[...]
        y = jnp.dot(ns + x, w1_ref[...], preferred_element_type=jnp.float32)
        y = y + jnp.dot(ns * x, w2_ref[...], preferred_element_type=jnp.float32)
        o_ref[...] = jnp.where(y >= 0, y, 0.2 * y)

    return pl.pallas_call(
        body,
        grid=(N // ROW_BLOCK,),
        in_specs=[
            pl.BlockSpec((2, ROW_BLOCK, D), lambda i: (0, i, 0)),
            pl.BlockSpec((ROW_BLOCK, D), lambda i: (i, 0)),
            pl.BlockSpec((D, D), lambda i: (0, 0)),
            pl.BlockSpec((D, D), lambda i: (0, 0)),
        ],
        out_specs=pl.BlockSpec((ROW_BLOCK, D), lambda i: (i, 0)),
        out_shape=jax.ShapeDtypeStruct((N, D), jnp.float32),
    )(parts, features, W1, W2)


def kernel(features, edge_index, W1, W2):
    ei = edge_index.reshape(2, NCROWS, CHUNK)
    parts = _sc_neigh_sum(features, ei)
    return _tc_finish(parts.reshape(NC, N, D), features, W1, W2)


# speedup vs baseline: 1.0878x; 1.0878x over previous
import functools

import jax
import jax.numpy as jnp
from jax import lax
from jax.experimental import pallas as pl
from jax.experimental.pallas import tpu as pltpu
from jax.experimental.pallas import tpu_sc as plsc

N = 10000
E = 320000
D = 128

NC = 2
NS = 16
NW = NC * NS
CHUNK = 125
NSLOT = 2
NCROWS = E // CHUNK
GPW = NCROWS // NW
BLK = 8
NBLOCKS = GPW // BLK

RPT = 624
TAIL = N - NS * RPT
ZROWS = 104

ROW_BLOCK = 2000


def _sc_neigh_sum(features, ei):
    mesh = plsc.VectorSubcoreMesh(
        core_axis_name="c", subcore_axis_name="s",
        num_cores=NC, num_subcores=NS)

    @functools.partial(
        pl.kernel,
        out_type=jax.ShapeDtypeStruct((NC * N, D), jnp.float32),
        mesh=mesh,
        scratch_types=[
            pltpu.VMEM((2, BLK, CHUNK), jnp.int32),
            pltpu.VMEM((2, BLK, CHUNK), jnp.int32),
            pltpu.VMEM((NSLOT, CHUNK, D), jnp.float32),
            pltpu.VMEM_SHARED((N, D), jnp.float32),
            pltpu.SemaphoreType.DMA,
            pltpu.SemaphoreType.DMA,
        ],
    )
    def k(feat_hbm, ei_hbm, out_hbm,
          sbuf, dbuf, rows_v, accum, gsem, ssem):
        cid = lax.axis_index("c")
        sid = lax.axis_index("s")
        wid = sid * NC + cid

        def zrow(i, carry):
            for c in range(D // 16):
                rows_v[0, i, pl.ds(c * 16, 16)] = jnp.zeros((16,),
                                                            jnp.float32)
            return carry

        lax.fori_loop(0, ZROWS, zrow, 0)
        r0 = pl.multiple_of(sid * RPT, 8)
        zsrc = rows_v.at[0, pl.ds(0, ZROWS)]
        for q in range(RPT // ZROWS):
            pltpu.async_copy(zsrc, accum.at[pl.ds(r0 + q * ZROWS, ZROWS)],
                             ssem)

        @pl.when(sid == 0)
        def _():
            pltpu.async_copy(rows_v.at[0, pl.ds(0, TAIL)],
                             accum.at[pl.ds(NS * RPT, TAIL)], ssem)

        for q in range(RPT // ZROWS):
            pltpu.make_async_copy(zsrc,
                                  accum.at[pl.ds(r0 + q * ZROWS, ZROWS)],
                                  ssem).wait()

        @pl.when(sid == 0)
        def _():
            pltpu.make_async_copy(rows_v.at[0, pl.ds(0, TAIL)],
                                  accum.at[pl.ds(NS * RPT, TAIL)],
                                  ssem).wait()

        plsc.subcore_barrier()

        base = wid * GPW

        def fire_gather(s, bsel, q):
            pltpu.async_copy(feat_hbm.at[sbuf.at[bsel, q]],
                             rows_v.at[s], gsem)

        def drain_gather(s):
            pltpu.make_async_copy(feat_hbm.at[sbuf.at[0, 0]],
                                  rows_v.at[s], gsem).wait()

        def fire_scatter(s, bsel, q):
            pltpu.async_copy(rows_v.at[s], accum.at[dbuf.at[bsel, q]],
                             ssem, add=True)

        def drain_scatter(s):
            pltpu.make_async_copy(rows_v.at[s], accum.at[dbuf.at[0, 0]],
                                  ssem).wait()

        pltpu.sync_copy(ei_hbm.at[0, pl.ds(pl.multiple_of(base, 8), BLK)],
                        sbuf.at[0])
        pltpu.sync_copy(ei_hbm.at[1, pl.ds(pl.multiple_of(base, 8), BLK)],
                        dbuf.at[0])
        for q in range(NSLOT):
            fire_gather(q, 0, q)

        def body(h, carry):
            nb = h + 1

            @pl.when(nb < NBLOCKS)
            def _():
                bsel_n = lax.rem(nb, 2)
                roff = pl.multiple_of(base + nb * BLK, 8)
                pltpu.sync_copy(ei_hbm.at[0, pl.ds(roff, BLK)],
                                sbuf.at[bsel_n])
                pltpu.sync_copy(ei_hbm.at[1, pl.ds(roff, BLK)],
                                dbuf.at[bsel_n])

            bcur = lax.rem(h, 2)
            bnext = lax.rem(h + 1, 2)
            for qq in range(BLK):
                s = qq % NSLOT
                drain_gather(s)
                fire_scatter(s, bcur, qq)
                drain_scatter(s)
                nxt_q = (qq + NSLOT) % BLK
                nxt_b = bcur if qq < BLK - NSLOT else bnext

                @pl.when(BLK * h + qq + NSLOT < GPW)
                def _(s=s, nxt_b=nxt_b, nxt_q=nxt_q):
                    fire_gather(s, nxt_b, nxt_q)

            return carry

        lax.fori_loop(0, NBLOCKS, body, 0)
        plsc.subcore_barrier()

        w0 = pl.multiple_of(cid * N + sid * RPT, 8)
        pltpu.sync_copy(accum.at[pl.ds(r0, RPT)], out_hbm.at[pl.ds(w0, RPT)])

        @pl.when(sid == 0)
        def _():
            pltpu.sync_copy(accum.at[pl.ds(NS * RPT, TAIL)],
                            out_hbm.at[pl.ds(cid * N + NS * RPT, TAIL)])

    return k(features, ei)


def _tc_finish(parts, features, W1, W2):
    def body(p_ref, x_ref, w1_ref, w2_ref, o_ref):
        ns = p_ref[0] + p_ref[1]
        x = x_ref[...]
        y = jnp.dot(ns + x, w1_ref[...], preferred_element_type=jnp.float32)
        y = y + jnp.dot(ns * x, w2_ref[...], preferred_element_type=jnp.float32)
        o_ref[...] = jnp.where(y >= 0, y, 0.2 * y)

    return pl.pallas_call(
        body,
        grid=(N // ROW_BLOCK,),
        in_specs=[
            pl.BlockSpec((2, ROW_BLOCK, D), lambda i: (0, i, 0)),
            pl.BlockSpec((ROW_BLOCK, D), lambda i: (i, 0)),
            pl.BlockSpec((D, D), lambda i: (0, 0)),
            pl.BlockSpec((D, D), lambda i: (0, 0)),
        ],
        out_specs=pl.BlockSpec((ROW_BLOCK, D), lambda i: (i, 0)),
        out_shape=jax.ShapeDtypeStruct((N, D), jnp.float32),
    )(parts, features, W1, W2)


def kernel(features, edge_index, W1, W2):
    ei = edge_index.reshape(2, NCROWS, CHUNK)
    parts = _sc_neigh_sum(features, ei)
    return _tc_finish(parts.reshape(NC, N, D), features, W1, W2)


# speedup vs baseline: 1.0901x; 1.0021x over previous
import functools

import jax
import jax.numpy as jnp
from jax import lax
from jax.experimental import pallas as pl
from jax.experimental.pallas import tpu as pltpu
from jax.experimental.pallas import tpu_sc as plsc

N = 10000
E = 320000
D = 128

NC = 2
NS = 16
NW = NC * NS
CHUNK = 125
NSLOT = 2
NCROWS = E // CHUNK
GPW = NCROWS // NW
BLK = 8
NBLOCKS = GPW // BLK

RPT = 624
TAIL = N - NS * RPT
ZROWS = 104

ROW_BLOCK = 2000


def _sc_neigh_sum(features, ei):
    mesh = plsc.VectorSubcoreMesh(
        core_axis_name="c", subcore_axis_name="s",
        num_cores=NC, num_subcores=NS)

    @functools.partial(
        pl.kernel,
        out_type=jax.ShapeDtypeStruct((NC, N, D), jnp.float32),
        mesh=mesh,
        scratch_types=[
            pltpu.VMEM((2, BLK, CHUNK), jnp.int32),
            pltpu.VMEM((2, BLK, CHUNK), jnp.int32),
            pltpu.VMEM((NSLOT, CHUNK, D), jnp.float32),
            pltpu.VMEM_SHARED((N, D), jnp.float32),
            pltpu.SemaphoreType.DMA,
            pltpu.SemaphoreType.DMA,
        ],
    )
    def k(feat_hbm, ei_hbm, out_hbm,
          sbuf, dbuf, rows_v, accum, gsem, ssem):
        cid = lax.axis_index("c")
        sid = lax.axis_index("s")
        wid = sid * NC + cid

        def zrow(i, carry):
            for c in range(D // 16):
                rows_v[0, i, pl.ds(c * 16, 16)] = jnp.zeros((16,),
                                                            jnp.float32)
            return carry

        lax.fori_loop(0, ZROWS, zrow, 0)
        r0 = pl.multiple_of(sid * RPT, 8)
        zsrc = rows_v.at[0, pl.ds(0, ZROWS)]
        for q in range(RPT // ZROWS):
            pltpu.async_copy(zsrc, accum.at[pl.ds(r0 + q * ZROWS, ZROWS)],
                             ssem)

        @pl.when(sid == 0)
        def _():
            pltpu.async_copy(rows_v.at[0, pl.ds(0, TAIL)],
                             accum.at[pl.ds(NS * RPT, TAIL)], ssem)

        for q in range(RPT // ZROWS):
            pltpu.make_async_copy(zsrc,
                                  accum.at[pl.ds(r0 + q * ZROWS, ZROWS)],
                                  ssem).wait()

        @pl.when(sid == 0)
        def _():
            pltpu.make_async_copy(rows_v.at[0, pl.ds(0, TAIL)],
                                  accum.at[pl.ds(NS * RPT, TAIL)],
                                  ssem).wait()

        plsc.subcore_barrier()

        base = wid * GPW

        def fire_gather(s, bsel, q):
            pltpu.async_copy(feat_hbm.at[sbuf.at[bsel, q]],
                             rows_v.at[s], gsem)

        def drain_gather(s):
            pltpu.make_async_copy(feat_hbm.at[sbuf.at[0, 0]],
                                  rows_v.at[s], gsem).wait()

        def fire_scatter(s, bsel, q):
            pltpu.async_copy(rows_v.at[s], accum.at[dbuf.at[bsel, q]],
                             ssem, add=True)

        def drain_scatter(s):
            pltpu.make_async_copy(rows_v.at[s], accum.at[dbuf.at[0, 0]],
                                  ssem).wait()

        pltpu.sync_copy(ei_hbm.at[0, pl.ds(pl.multiple_of(base, 8), BLK)],
                        sbuf.at[0])
        pltpu.sync_copy(ei_hbm.at[1, pl.ds(pl.multiple_of(base, 8), BLK)],
                        dbuf.at[0])
        for q in range(NSLOT):
            fire_gather(q, 0, q)

        def body(h, carry):
            nb = h + 1

            @pl.when(nb < NBLOCKS)
            def _():
                bsel_n = lax.rem(nb, 2)
                roff = pl.multiple_of(base + nb * BLK, 8)
                pltpu.sync_copy(ei_hbm.at[0, pl.ds(roff, BLK)],
                                sbuf.at[bsel_n])
                pltpu.sync_copy(ei_hbm.at[1, pl.ds(roff, BLK)],
                                dbuf.at[bsel_n])

            bcur = lax.rem(h, 2)
            bnext = lax.rem(h + 1, 2)
            for qq in range(BLK):
                s = qq % NSLOT
                drain_gather(s)
                fire_scatter(s, bcur, qq)
                drain_scatter(s)
                nxt_q = (qq + NSLOT) % BLK
                nxt_b = bcur if qq < BLK - NSLOT else bnext

                @pl.when(BLK * h + qq + NSLOT < GPW)
                def _(s=s, nxt_b=nxt_b, nxt_q=nxt_q):
                    fire_gather(s, nxt_b, nxt_q)

            return carry

        lax.fori_loop(0, NBLOCKS, body, 0)
        plsc.subcore_barrier()

        pltpu.sync_copy(accum.at[pl.ds(r0, RPT)],
                        out_hbm.at[cid, pl.ds(r0, RPT)])

        @pl.when(sid == 0)
        def _():
            pltpu.sync_copy(accum.at[pl.ds(NS * RPT, TAIL)],
                            out_hbm.at[cid, pl.ds(NS * RPT, TAIL)])

    return k(features, ei)


def _tc_finish(parts, features, W1, W2):
    def body(p_ref, x_ref, w1_ref, w2_ref, o_ref):
        ns = p_ref[0] + p_ref[1]
        x = x_ref[...]
        y = jnp.dot(ns + x, w1_ref[...], preferred_element_type=jnp.float32)
        y = y + jnp.dot(ns * x, w2_ref[...], preferred_element_type=jnp.float32)
        o_ref[...] = jnp.where(y >= 0, y, 0.2 * y)

    return pl.pallas_call(
        body,
        grid=(N // ROW_BLOCK,),
        in_specs=[
            pl.BlockSpec((2, ROW_BLOCK, D), lambda i: (0, i, 0)),
            pl.BlockSpec((ROW_BLOCK, D), lambda i: (i, 0)),
            pl.BlockSpec((D, D), lambda i: (0, 0)),
            pl.BlockSpec((D, D), lambda i: (0, 0)),
        ],
        out_specs=pl.BlockSpec((ROW_BLOCK, D), lambda i: (i, 0)),
        out_shape=jax.ShapeDtypeStruct((N, D), jnp.float32),
    )(parts, features, W1, W2)


def kernel(features, edge_index, W1, W2):
    ei = edge_index.reshape(2, NCROWS, CHUNK)
    parts = _sc_neigh_sum(features, ei)
    return _tc_finish(parts, features, W1, W2)


# speedup vs baseline: 1.0976x; 1.0069x over previous
import functools

import jax
import jax.numpy as jnp
from jax import lax
from jax.experimental import pallas as pl
from jax.experimental.pallas import tpu as pltpu
from jax.experimental.pallas import tpu_sc as plsc

N = 10000
E = 320000
D = 128

NC = 2
NS = 16
NW = NC * NS
CHUNK = 125
NSLOT = 2
NCROWS = E // CHUNK
GPW = NCROWS // NW
BLK = 8
NBLOCKS = GPW // BLK

RPT = 624
TAIL = N - NS * RPT
ZROWS = 104

ROW_BLOCK = 5000


def _sc_neigh_sum(features, ei):
    mesh = plsc.VectorSubcoreMesh(
        core_axis_name="c", subcore_axis_name="s",
        num_cores=NC, num_subcores=NS)

    @functools.partial(
        pl.kernel,
        out_type=jax.ShapeDtypeStruct((NC, N, D), jnp.float32),
        mesh=mesh,
        scratch_types=[
            pltpu.VMEM((2, BLK, CHUNK), jnp.int32),
            pltpu.VMEM((2, BLK, CHUNK), jnp.int32),
            pltpu.VMEM((NSLOT, CHUNK, D), jnp.float32),
            pltpu.VMEM_SHARED((N, D), jnp.float32),
            pltpu.SemaphoreType.DMA,
            pltpu.SemaphoreType.DMA,
        ],
    )
    def k(feat_hbm, ei_hbm, out_hbm,
          sbuf, dbuf, rows_v, accum, gsem, ssem):
        cid = lax.axis_index("c")
        sid = lax.axis_index("s")
        wid = sid * NC + cid

        def zrow(i, carry):
            for c in range(D // 16):
                rows_v[0, i, pl.ds(c * 16, 16)] = jnp.zeros((16,),
                                                            jnp.float32)
            return carry

        lax.fori_loop(0, ZROWS, zrow, 0)
        r0 = pl.multiple_of(sid * RPT, 8)
        zsrc = rows_v.at[0, pl.ds(0, ZROWS)]
        for q in range(RPT // ZROWS):
            pltpu.async_copy(zsrc, accum.at[pl.ds(r0 + q * ZROWS, ZROWS)],
                             ssem)

        @pl.when(sid == 0)
        def _():
            pltpu.async_copy(rows_v.at[0, pl.ds(0, TAIL)],
                             accum.at[pl.ds(NS * RPT, TAIL)], ssem)

        for q in range(RPT // ZROWS):
            pltpu.make_async_copy(zsrc,
                                  accum.at[pl.ds(r0 + q * ZROWS, ZROWS)],
                                  ssem).wait()

        @pl.when(sid == 0)
        def _():
            pltpu.make_async_copy(rows_v.at[0, pl.ds(0, TAIL)],
                                  accum.at[pl.ds(NS * RPT, TAIL)],
                                  ssem).wait()

        plsc.subcore_barrier()

        base = wid * GPW

        def fire_gather(s, bsel, q):
            pltpu.async_copy(feat_hbm.at[sbuf.at[bsel, q]],
                             rows_v.at[s], gsem)

        def drain_gather(s):
            pltpu.make_async_copy(feat_hbm.at[sbuf.at[0, 0]],
                                  rows_v.at[s], gsem).wait()

        def fire_scatter(s, bsel, q):
            pltpu.async_copy(rows_v.at[s], accum.at[dbuf.at[bsel, q]],
                             ssem, add=True)

        def drain_scatter(s):
            pltpu.make_async_copy(rows_v.at[s], accum.at[dbuf.at[0, 0]],
                                  ssem).wait()

        pltpu.sync_copy(ei_hbm.at[0, pl.ds(pl.multiple_of(base, 8), BLK)],
                        sbuf.at[0])
        pltpu.sync_copy(ei_hbm.at[1, pl.ds(pl.multiple_of(base, 8), BLK)],
                        dbuf.at[0])
        for q in range(NSLOT):
            fire_gather(q, 0, q)

        def body(h, carry):
            nb = h + 1

            @pl.when(nb < NBLOCKS)
            def _():
                bsel_n = lax.rem(nb, 2)
                roff = pl.multiple_of(base + nb * BLK, 8)
                pltpu.sync_copy(ei_hbm.at[0, pl.ds(roff, BLK)],
                                sbuf.at[bsel_n])
                pltpu.sync_copy(ei_hbm.at[1, pl.ds(roff, BLK)],
                                dbuf.at[bsel_n])

            bcur = lax.rem(h, 2)
            bnext = lax.rem(h + 1, 2)
            for qq in range(BLK):
                s = qq % NSLOT
                drain_gather(s)
                fire_scatter(s, bcur, qq)
                drain_scatter(s)
                nxt_q = (qq + NSLOT) % BLK
                nxt_b = bcur if qq < BLK - NSLOT else bnext

                @pl.when(BLK * h + qq + NSLOT < GPW)
                def _(s=s, nxt_b=nxt_b, nxt_q=nxt_q):
                    fire_gather(s, nxt_b, nxt_q)

            return carry

        lax.fori_loop(0, NBLOCKS, body, 0)
        plsc.subcore_barrier()

        pltpu.sync_copy(accum.at[pl.ds(r0, RPT)],
                        out_hbm.at[cid, pl.ds(r0, RPT)])

        @pl.when(sid == 0)
        def _():
            pltpu.sync_copy(accum.at[pl.ds(NS * RPT, TAIL)],
                            out_hbm.at[cid, pl.ds(NS * RPT, TAIL)])

    return k(features, ei)


def _tc_finish(parts, features, W1, W2):
    def body(p_ref, x_ref, w1_ref, w2_ref, o_ref):
        ns = p_ref[0] + p_ref[1]
        x = x_ref[...]
        y = jnp.dot(ns + x, w1_ref[...], preferred_element_type=jnp.float32)
        y = y + jnp.dot(ns * x, w2_ref[...], preferred_element_type=jnp.float32)
        o_ref[...] = jnp.where(y >= 0, y, 0.2 * y)

    return pl.pallas_call(
        body,
        grid=(N // ROW_BLOCK,),
        in_specs=[
            pl.BlockSpec((2, ROW_BLOCK, D), lambda i: (0, i, 0)),
            pl.BlockSpec((ROW_BLOCK, D), lambda i: (i, 0)),
            pl.BlockSpec((D, D), lambda i: (0, 0)),
            pl.BlockSpec((D, D), lambda i: (0, 0)),
        ],
        out_specs=pl.BlockSpec((ROW_BLOCK, D), lambda i: (i, 0)),
        out_shape=jax.ShapeDtypeStruct((N, D), jnp.float32),
    )(parts, features, W1, W2)


def kernel(features, edge_index, W1, W2):
    ei = edge_index.reshape(2, NCROWS, CHUNK)
    parts = _sc_neigh_sum(features, ei)
    return _tc_finish(parts, features, W1, W2)
